# baseline (speedup 1.0000x reference)
import functools

import jax
import jax.numpy as jnp
from jax import lax
from jax.experimental import pallas as pl
from jax.experimental.pallas import tpu as pltpu
from jax.experimental.pallas import tpu_sc as plsc

TOPK = 5
BANK_BLOCK = 4096
NEG = float("-inf")
IMAX = jnp.iinfo(jnp.int32).max


BIGID = 3e7


def _extract_top5(s, col, ids_positional=True):
    bq, n = s.shape
    ng = n // 128
    lane = col[:, :128]
    vals, ids = [], []
    for t in range(TOPK):
        accv = s[:, 0:128]
        acci = jnp.zeros((bq, 128), jnp.float32) if ids_positional else lane
        for g in range(1, ng):
            sg = s[:, g * 128:(g + 1) * 128]
            if ids_positional:
                c = sg > accv
                acci = jnp.where(c, jnp.float32(g), acci)
            else:
                cg = col[:, g * 128:(g + 1) * 128]
                c = (sg > accv) | ((sg == accv) & (cg < acci))
                acci = jnp.where(c, cg, acci)
            accv = jnp.where(c, sg, accv)
        m = jnp.max(accv, axis=1, keepdims=True)
        cand = acci * 128.0 + lane if ids_positional else acci
        w = jnp.min(jnp.where(accv == m, cand, BIGID), axis=1, keepdims=True)
        vals.append(m)
        ids.append(w)
        if t < TOPK - 1:
            s = jnp.where(col == w, NEG, s)
    return vals, ids


def _block_body(k_total, z_ref, zn_ref, bank_ref, bnorm_ref, cv_ref, ci_ref):
    bq = z_ref.shape[0]
    j = pl.program_id(0)
    zn = z_ref[...] / zn_ref[...]
    bn = bank_ref[...] / bnorm_ref[...]
    s = lax.dot_general(zn, bn, (((1,), (1,)), ((), ())),
                        preferred_element_type=jnp.float32)
    col = lax.broadcasted_iota(jnp.int32, s.shape, 1).astype(jnp.float32)
    bound = (k_total - j * BANK_BLOCK).astype(jnp.float32)
    s = jnp.where(col < bound, s, NEG)

    bvals, bids = _extract_top5(s, col)
    off = (j * BANK_BLOCK).astype(jnp.float32)
    cv_ref[0] = jnp.concatenate(
        bvals + [jnp.full((bq, 3), NEG, jnp.float32)], axis=1)
    ci_ref[0] = jnp.concatenate(
        [w + off for w in bids] + [jnp.full((bq, 3), BIGID, jnp.float32)], axis=1)


def _merge_body(cv_ref, ci_ref, idx_ref, sim_ref):
    vals, ids = _extract_top5(cv_ref[...], ci_ref[...], ids_positional=False)
    sim_ref[...] = jnp.concatenate(vals, axis=1)
    idx_ref[...] = jnp.concatenate(ids, axis=1).astype(jnp.int32)


def _cosine_topk(z, bank):
    bq, d = z.shape
    k_total = bank.shape[0]
    nb = (k_total + BANK_BLOCK - 1) // BANK_BLOCK
    nb_full = k_total // BANK_BLOCK
    eps = 1e-12
    znorm = jnp.maximum(jnp.linalg.norm(z, axis=-1, keepdims=True), eps)
    bnorm = jnp.maximum(jnp.linalg.norm(bank, axis=-1, keepdims=True), eps)
    cv, ci = pl.pallas_call(
        functools.partial(_block_body, k_total),
        grid=(nb,),
        in_specs=[
            pl.BlockSpec((bq, d), lambda j: (0, 0)),
            pl.BlockSpec((bq, 1), lambda j: (0, 0)),
            pl.BlockSpec((BANK_BLOCK, d), lambda j: (j, 0)),
            pl.BlockSpec((BANK_BLOCK, 1), lambda j: (j, 0)),
        ],
        out_specs=[
            pl.BlockSpec((1, bq, 8), lambda j: (j, 0, 0)),
            pl.BlockSpec((1, bq, 8), lambda j: (j, 0, 0)),
        ],
        out_shape=[
            jax.ShapeDtypeStruct((nb, bq, 8), jnp.float32),
            jax.ShapeDtypeStruct((nb, bq, 8), jnp.float32),
        ],
    )(z, znorm, bank, bnorm)

    ncand = nb * 8
    npad = ((ncand + 127) // 128) * 128
    cv = cv.transpose(1, 0, 2).reshape(bq, ncand)
    ci = ci.transpose(1, 0, 2).reshape(bq, ncand)
    if npad != ncand:
        cv = jnp.pad(cv, ((0, 0), (0, npad - ncand)), constant_values=NEG)
        ci = jnp.pad(ci, ((0, 0), (0, npad - ncand)), constant_values=float(BIGID))
    return pl.pallas_call(
        _merge_body,
        out_shape=[
            jax.ShapeDtypeStruct((bq, TOPK), jnp.int32),
            jax.ShapeDtypeStruct((bq, TOPK), jnp.float32),
        ],
    )(cv, ci)


def _gather_labels_sc(table, idx_flat):
    info = plsc.get_sparse_core_info()
    nw = info.num_cores * info.num_subcores
    b_tot = idx_flat.shape[0]
    assert b_tot % (8 * nw) == 0
    b_per_w = b_tot // nw
    mesh = plsc.VectorSubcoreMesh(core_axis_name="c", subcore_axis_name="s")

    @functools.partial(
        pl.kernel,
        mesh=mesh,
        out_type=jax.ShapeDtypeStruct((b_tot,), jnp.int32),
        scratch_types=[
            pltpu.VMEM((b_per_w,), jnp.int32),
            pltpu.VMEM((b_per_w,), jnp.int32),
            pltpu.SemaphoreType.DMA,
        ],
    )
    def _gather(table_hbm, idx_hbm, out_hbm, idx_v, rows_v, sem):
        wid = lax.axis_index("s") * info.num_cores + lax.axis_index("c")
        base = wid * b_per_w
        pltpu.sync_copy(idx_hbm.at[pl.ds(base, b_per_w)], idx_v)
        pltpu.async_copy(table_hbm.at[idx_v], rows_v, sem).wait()
        pltpu.sync_copy(rows_v, out_hbm.at[pl.ds(base, b_per_w)])

    return _gather(table, idx_flat)


def kernel(z, lab_bank, lab_labels, topk):
    idx, sim = _cosine_topk(z, lab_bank)
    labels = _gather_labels_sc(lab_labels, idx.reshape(-1)).reshape(idx.shape)
    return idx, sim, labels


# speedup vs baseline: 1.0118x; 1.0118x over previous
import functools

import jax
import jax.numpy as jnp
from jax import lax
from jax.experimental import pallas as pl
from jax.experimental.pallas import tpu as pltpu
from jax.experimental.pallas import tpu_sc as plsc

TOPK = 5
BANK_BLOCK = 4096
NEG = float("-inf")
IMAX = jnp.iinfo(jnp.int32).max


BIGID = 3e7


def _extract_top5(s, col, ids_positional=True):
    bq, n = s.shape
    ng = n // 128
    lane = col[:, :128]
    vals, ids = [], []
    for t in range(TOPK):
        accv = s[:, 0:128]
        acci = jnp.zeros((bq, 128), jnp.float32) if ids_positional else lane
        for g in range(1, ng):
            sg = s[:, g * 128:(g + 1) * 128]
            if ids_positional:
                c = sg > accv
                acci = jnp.where(c, jnp.float32(g), acci)
            else:
                cg = col[:, g * 128:(g + 1) * 128]
                c = (sg > accv) | ((sg == accv) & (cg < acci))
                acci = jnp.where(c, cg, acci)
            accv = jnp.where(c, sg, accv)
        m = jnp.max(accv, axis=1, keepdims=True)
        cand = acci * 128.0 + lane if ids_positional else acci
        w = jnp.min(jnp.where(accv == m, cand, BIGID), axis=1, keepdims=True)
        vals.append(m)
        ids.append(w)
        if t < TOPK - 1:
            s = jnp.where(col == w, NEG, s)
    return vals, ids


def _block_core(zn_ref, bank_ref, bnorm_ref, cv_ref, ci_ref, off, bound):
    bq = zn_ref.shape[0]
    zn = zn_ref[...]
    bn = bank_ref[...] / bnorm_ref[...]
    s = lax.dot_general(zn, bn, (((1,), (1,)), ((), ())),
                        preferred_element_type=jnp.float32)
    col = lax.broadcasted_iota(jnp.int32, s.shape, 1).astype(jnp.float32)
    if bound is not None:
        s = jnp.where(col < bound, s, NEG)

    bvals, bids = _extract_top5(s, col)
    cv_ref[0] = jnp.concatenate(
        bvals + [jnp.full((bq, 3), NEG, jnp.float32)], axis=1)
    ci_ref[0] = jnp.concatenate(
        [w + off for w in bids] + [jnp.full((bq, 3), BIGID, jnp.float32)], axis=1)


def _main_body(zn_ref, bank_ref, bnorm_ref, cv_ref, ci_ref):
    off = (pl.program_id(0) * BANK_BLOCK).astype(jnp.float32)
    _block_core(zn_ref, bank_ref, bnorm_ref, cv_ref, ci_ref, off, None)


def _tail_body(k_total, nb_full, zn_ref, bank_ref, bnorm_ref, cv_ref, ci_ref):
    base = nb_full * BANK_BLOCK
    _block_core(zn_ref, bank_ref, bnorm_ref, cv_ref, ci_ref,
                jnp.float32(base), float(k_total - base))


def _merge_body(cv_ref, ci_ref, idx_ref, sim_ref):
    vals, ids = _extract_top5(cv_ref[...], ci_ref[...], ids_positional=False)
    sim_ref[...] = jnp.concatenate(vals, axis=1)
    idx_ref[...] = jnp.concatenate(ids, axis=1).astype(jnp.int32)


def _cosine_topk(z, bank):
    bq, d = z.shape
    k_total = bank.shape[0]
    nb = (k_total + BANK_BLOCK - 1) // BANK_BLOCK
    nb_full = k_total // BANK_BLOCK
    eps = 1e-12
    zn = z / jnp.maximum(jnp.linalg.norm(z, axis=-1, keepdims=True), eps)
    bnorm = jnp.maximum(jnp.linalg.norm(bank, axis=-1, keepdims=True), eps)
    cv, ci = pl.pallas_call(
        _main_body,
        grid=(nb_full,),
        in_specs=[
            pl.BlockSpec((bq, d), lambda j: (0, 0)),
            pl.BlockSpec((BANK_BLOCK, d), lambda j: (j, 0)),
            pl.BlockSpec((BANK_BLOCK, 1), lambda j: (j, 0)),
        ],
        out_specs=[
            pl.BlockSpec((1, bq, 8), lambda j: (j, 0, 0)),
            pl.BlockSpec((1, bq, 8), lambda j: (j, 0, 0)),
        ],
        out_shape=[
            jax.ShapeDtypeStruct((nb_full, bq, 8), jnp.float32),
            jax.ShapeDtypeStruct((nb_full, bq, 8), jnp.float32),
        ],
    )(zn, bank, bnorm)
    if nb != nb_full:
        tv, ti = pl.pallas_call(
            functools.partial(_tail_body, k_total, nb_full),
            grid=(1,),
            in_specs=[
                pl.BlockSpec((bq, d), lambda j: (0, 0)),
                pl.BlockSpec((BANK_BLOCK, d), lambda j: (nb_full, 0)),
                pl.BlockSpec((BANK_BLOCK, 1), lambda j: (nb_full, 0)),
            ],
            out_specs=[
                pl.BlockSpec((1, bq, 8), lambda j: (0, 0, 0)),
                pl.BlockSpec((1, bq, 8), lambda j: (0, 0, 0)),
            ],
            out_shape=[
                jax.ShapeDtypeStruct((1, bq, 8), jnp.float32),
                jax.ShapeDtypeStruct((1, bq, 8), jnp.float32),
            ],
        )(zn, bank, bnorm)
        cv = jnp.concatenate([cv, tv], axis=0)
        ci = jnp.concatenate([ci, ti], axis=0)

    ncand = nb * 8
    npad = ((ncand + 127) // 128) * 128
    cv = cv.transpose(1, 0, 2).reshape(bq, ncand)
    ci = ci.transpose(1, 0, 2).reshape(bq, ncand)
    if npad != ncand:
        cv = jnp.pad(cv, ((0, 0), (0, npad - ncand)), constant_values=NEG)
        ci = jnp.pad(ci, ((0, 0), (0, npad - ncand)), constant_values=float(BIGID))
    return pl.pallas_call(
        _merge_body,
        out_shape=[
            jax.ShapeDtypeStruct((bq, TOPK), jnp.int32),
            jax.ShapeDtypeStruct((bq, TOPK), jnp.float32),
        ],
    )(cv, ci)


def _gather_labels_sc(table, idx_flat):
    info = plsc.get_sparse_core_info()
    nw = info.num_cores * info.num_subcores
    b_tot = idx_flat.shape[0]
    assert b_tot % (8 * nw) == 0
    b_per_w = b_tot // nw
    mesh = plsc.VectorSubcoreMesh(core_axis_name="c", subcore_axis_name="s")

    @functools.partial(
        pl.kernel,
        mesh=mesh,
        out_type=jax.ShapeDtypeStruct((b_tot,), jnp.int32),
        scratch_types=[
            pltpu.VMEM((b_per_w,), jnp.int32),
            pltpu.VMEM((b_per_w,), jnp.int32),
            pltpu.SemaphoreType.DMA,
        ],
    )
    def _gather(table_hbm, idx_hbm, out_hbm, idx_v, rows_v, sem):
        wid = lax.axis_index("s") * info.num_cores + lax.axis_index("c")
        base = wid * b_per_w
        pltpu.sync_copy(idx_hbm.at[pl.ds(base, b_per_w)], idx_v)
        pltpu.async_copy(table_hbm.at[idx_v], rows_v, sem).wait()
        pltpu.sync_copy(rows_v, out_hbm.at[pl.ds(base, b_per_w)])

    return _gather(table, idx_flat)


def kernel(z, lab_bank, lab_labels, topk):
    idx, sim = _cosine_topk(z, lab_bank)
    labels = _gather_labels_sc(lab_labels, idx.reshape(-1)).reshape(idx.shape)
    return idx, sim, labels
